# baseline (speedup 1.0000x reference)
import functools

import jax
import jax.numpy as jnp
from jax import lax
from jax.experimental import pallas as pl
from jax.experimental.pallas import tpu as pltpu
from jax.experimental.pallas import tpu_sc as plsc

_B = 16384
_D = 16
_NW = 32
_BPW = _B // _NW
_K = 8
_NG = _BPW // 16


def _body(user_hbm, item_hbm, t0_hbm, t1_hbm, t2_hbm, t3_hbm, out_hbm,
          uidx_v, iidx_v, panel_v, ring_v, sem):
    wid = lax.axis_index("s") * 2 + lax.axis_index("c")
    base = wid * _BPW

    pltpu.sync_copy(user_hbm.at[pl.ds(base, _BPW)], uidx_v)
    pltpu.sync_copy(item_hbm.at[pl.ds(base, _BPW)], iidx_v)

    c_iota = lax.iota(jnp.int32, 16)
    tables = (t0_hbm, t1_hbm, t2_hbm, t3_hbm)

    def group(g, carry):
        uvec = uidx_v[pl.ds(g * 16, 16)]
        ivec = iidx_v[pl.ds(g * 16, 16)]
        uoff = (uvec >> 7) << 7
        ioff = (ivec >> 7) << 7
        ulane = uvec & 127
        ilane = ivec & 127
        for half in range(2):
            copies = []
            for kk in range(_K):
                e = half * _K + kk
                for t in range(4):
                    off = uoff[e] if t % 2 == 0 else ioff[e]
                    copies.append(pltpu.async_copy(
                        tables[t].at[:, pl.ds(pl.multiple_of(off, 128), 128)],
                        ring_v.at[kk * 4 + t], sem))
            for c in copies:
                c.wait()
            for kk in range(_K):
                e = half * _K + kk
                j = g * 16 + e
                jv = jnp.full((16,), j, jnp.int32)
                for t in range(4):
                    lane = ulane[e] if t % 2 == 0 else ilane[e]
                    vec = plsc.load_gather(
                        ring_v.at[kk * 4 + t],
                        [c_iota, jnp.full((16,), lane, jnp.int32)])
                    plsc.store_scatter(panel_v, [t * 16 + c_iota, jv], vec)
        return carry

    lax.fori_loop(0, _NG, group, 0)
    pltpu.sync_copy(panel_v, out_hbm.at[:, pl.ds(base, _BPW)])


@jax.jit
def _run(user, item, t0, t1, t2, t3):
    mesh = plsc.VectorSubcoreMesh(core_axis_name="c", subcore_axis_name="s")
    k = functools.partial(
        pl.kernel,
        mesh=mesh,
        out_type=jax.ShapeDtypeStruct((4 * _D, _B), jnp.float32),
        scratch_types=[
            pltpu.VMEM((_BPW,), jnp.int32),
            pltpu.VMEM((_BPW,), jnp.int32),
            pltpu.VMEM((4 * _D, _BPW), jnp.float32),
            pltpu.VMEM((4 * _K, 16, 128), jnp.float32),
            pltpu.SemaphoreType.DMA,
        ],
        compiler_params=pltpu.CompilerParams(needs_layout_passes=False),
    )(_body)
    return k(user, item, t0, t1, t2, t3)


def kernel(user, item, MF_Embedding_User, MF_Embedding_Item,
           MLP_Embedding_User, MLP_Embedding_Item):
    out = _run(user, item, MF_Embedding_User.T, MF_Embedding_Item.T,
               MLP_Embedding_User.T, MLP_Embedding_Item.T)
    return out.T


# speedup vs baseline: 1.3568x; 1.3568x over previous
import functools

import jax
import jax.numpy as jnp
from jax import lax
from jax.experimental import pallas as pl
from jax.experimental.pallas import tpu as pltpu
from jax.experimental.pallas import tpu_sc as plsc

_B = 16384
_D = 16
_NW = 32
_BPW = _B // _NW
_W = 4
_NWAVE = _BPW // _W


def _body(user_hbm, item_hbm, t0_hbm, t1_hbm, t2_hbm, t3_hbm, out_hbm,
          uidx_v, iidx_v, panel_v, ring_v, usm, ism, sem):
    wid = lax.axis_index("s") * 2 + lax.axis_index("c")
    base = wid * _BPW

    pltpu.sync_copy(user_hbm.at[pl.ds(base, _BPW)], uidx_v)
    pltpu.sync_copy(item_hbm.at[pl.ds(base, _BPW)], iidx_v)

    c_iota = lax.iota(jnp.int32, 16)
    tables = (t0_hbm, t1_hbm, t2_hbm, t3_hbm)

    def spill(q, carry):
        uvec = uidx_v[pl.ds(q * 16, 16)]
        ivec = iidx_v[pl.ds(q * 16, 16)]
        for e in range(16):
            usm[q * 16 + e] = uvec[e]
            ism[q * 16 + e] = ivec[e]
        return carry

    lax.fori_loop(0, _BPW // 16, spill, 0)

    def issue(g, parity):
        for kk in range(_W):
            u = usm[g * _W + kk]
            i = ism[g * _W + kk]
            for t in range(4):
                idx = u if t % 2 == 0 else i
                off = pl.multiple_of((idx >> 7) << 7, 128)
                pltpu.async_copy(
                    tables[t].at[:, pl.ds(off, 128)],
                    ring_v.at[parity, kk * 4 + t], sem)

    def drain_and_extract(g, parity):
        for s in range(4 * _W):
            pltpu.make_async_copy(
                tables[0].at[:, pl.ds(0, 128)],
                ring_v.at[parity, s], sem).wait()
        for kk in range(_W):
            j = g * _W + kk
            u = usm[j]
            i = ism[j]
            jv = jnp.full((16,), j, jnp.int32)
            for t in range(4):
                lane = (u if t % 2 == 0 else i) & 127
                vec = plsc.load_gather(
                    ring_v.at[parity, kk * 4 + t],
                    [c_iota, jnp.full((16,), lane, jnp.int32)])
                plsc.store_scatter(panel_v, [t * 16 + c_iota, jv], vec)

    def wave(g, carry):
        parity = g & 1
        issue(g, parity)

        @pl.when(g > 0)
        def _():
            drain_and_extract(g - 1, 1 - parity)

        return carry

    lax.fori_loop(0, _NWAVE, wave, 0)
    drain_and_extract(_NWAVE - 1, (_NWAVE - 1) & 1)

    pltpu.sync_copy(panel_v, out_hbm.at[:, pl.ds(base, _BPW)])


@jax.jit
def _run(user, item, t0, t1, t2, t3):
    mesh = plsc.VectorSubcoreMesh(core_axis_name="c", subcore_axis_name="s")
    k = functools.partial(
        pl.kernel,
        mesh=mesh,
        out_type=jax.ShapeDtypeStruct((4 * _D, _B), jnp.float32),
        scratch_types=[
            pltpu.VMEM((_BPW,), jnp.int32),
            pltpu.VMEM((_BPW,), jnp.int32),
            pltpu.VMEM((4 * _D, _BPW), jnp.float32),
            pltpu.VMEM((2, 4 * _W, 16, 128), jnp.float32),
            pltpu.SMEM((_BPW,), jnp.int32),
            pltpu.SMEM((_BPW,), jnp.int32),
            pltpu.SemaphoreType.DMA,
        ],
        compiler_params=pltpu.CompilerParams(needs_layout_passes=False),
    )(_body)
    return k(user, item, t0, t1, t2, t3)


def kernel(user, item, MF_Embedding_User, MF_Embedding_Item,
           MLP_Embedding_User, MLP_Embedding_Item):
    out = _run(user, item, MF_Embedding_User.T, MF_Embedding_Item.T,
               MLP_Embedding_User.T, MLP_Embedding_Item.T)
    return out.T


# speedup vs baseline: 1.3608x; 1.0029x over previous
import functools

import jax
import jax.numpy as jnp
from jax import lax
from jax.experimental import pallas as pl
from jax.experimental.pallas import tpu as pltpu
from jax.experimental.pallas import tpu_sc as plsc

_B = 16384
_D = 16
_NW = 32
_BPW = _B // _NW
_W = 4
_NWAVE = _BPW // _W
_WPC = 128 // _W


def _body(user_hbm, item_hbm, t0_hbm, t1_hbm, t2_hbm, t3_hbm, out_hbm,
          uidx_v, iidx_v, panel_v, ring_v, usm, ism, sem, wsem):
    wid = lax.axis_index("s") * 2 + lax.axis_index("c")
    base = wid * _BPW

    pltpu.sync_copy(user_hbm.at[pl.ds(base, _BPW)], uidx_v)
    pltpu.sync_copy(item_hbm.at[pl.ds(base, _BPW)], iidx_v)

    c_iota = lax.iota(jnp.int32, 16)
    rows_t = [t * _D + c_iota for t in range(4)]
    tables = (t0_hbm, t1_hbm, t2_hbm, t3_hbm)

    def spill(q, carry):
        uvec = uidx_v[pl.ds(q * 16, 16)]
        ivec = iidx_v[pl.ds(q * 16, 16)]
        for e in range(16):
            usm[q * 16 + e] = uvec[e]
            ism[q * 16 + e] = ivec[e]
        return carry

    lax.fori_loop(0, _BPW // 16, spill, 0)

    def issue(g, par):
        for kk in range(_W):
            u = usm[g * _W + kk]
            i = ism[g * _W + kk]
            for t in range(4):
                idx = u if t % 2 == 0 else i
                off = pl.multiple_of((idx >> 7) << 7, 128)
                pltpu.async_copy(
                    tables[t].at[:, pl.ds(off, 128)],
                    ring_v.at[par, kk * 4 + t], sem)

    def drain_and_extract(e, par):
        for s in range(4 * _W):
            pltpu.make_async_copy(
                tables[0].at[:, pl.ds(0, 128)],
                ring_v.at[par, s], sem).wait()
        buf = (e // _WPC) & 1
        for kk in range(_W):
            j = e * _W + kk
            jcol = j & 127
            u = usm[j]
            i = ism[j]
            jv = jnp.full((16,), jcol, jnp.int32)
            for t in range(4):
                lane = (u if t % 2 == 0 else i) & 127
                vec = plsc.load_gather(
                    ring_v.at[par, kk * 4 + t],
                    [c_iota, jnp.full((16,), lane, jnp.int32)])
                plsc.store_scatter(panel_v.at[buf], [rows_t[t], jv], vec)

    def issue_writeback(c):
        off = pl.multiple_of(base + c * 128, 128)
        pltpu.async_copy(
            panel_v.at[c & 1], out_hbm.at[:, pl.ds(off, 128)], wsem)

    def drain_writeback():
        pltpu.make_async_copy(
            panel_v.at[0], out_hbm.at[:, pl.ds(0, 128)], wsem).wait()

    def wave(g, carry):
        @pl.when(g < _NWAVE)
        def _():
            issue(g, g % 3)

        @pl.when(g >= 2)
        def _():
            e = g - 2

            @pl.when((e % _WPC == 0) & (e >= 2 * _WPC))
            def _():
                drain_writeback()

            drain_and_extract(e, e % 3)

            @pl.when(e % _WPC == _WPC - 1)
            def _():
                issue_writeback(e // _WPC)

        return carry

    lax.fori_loop(0, _NWAVE + 2, wave, 0)
    drain_writeback()
    drain_writeback()


@jax.jit
def _run(user, item, t0, t1, t2, t3):
    mesh = plsc.VectorSubcoreMesh(core_axis_name="c", subcore_axis_name="s")
    k = functools.partial(
        pl.kernel,
        mesh=mesh,
        out_type=jax.ShapeDtypeStruct((4 * _D, _B), jnp.float32),
        scratch_types=[
            pltpu.VMEM((_BPW,), jnp.int32),
            pltpu.VMEM((_BPW,), jnp.int32),
            pltpu.VMEM((2, 4 * _D, 128), jnp.float32),
            pltpu.VMEM((3, 4 * _W, 16, 128), jnp.float32),
            pltpu.SMEM((_BPW,), jnp.int32),
            pltpu.SMEM((_BPW,), jnp.int32),
            pltpu.SemaphoreType.DMA,
            pltpu.SemaphoreType.DMA,
        ],
        compiler_params=pltpu.CompilerParams(needs_layout_passes=False),
    )(_body)
    return k(user, item, t0, t1, t2, t3)


def kernel(user, item, MF_Embedding_User, MF_Embedding_Item,
           MLP_Embedding_User, MLP_Embedding_Item):
    out = _run(user, item, MF_Embedding_User.T, MF_Embedding_Item.T,
               MLP_Embedding_User.T, MLP_Embedding_Item.T)
    return out.T
